# baseline (speedup 1.0000x reference)
import functools
import math

import numpy as np
import jax
import jax.numpy as jnp
from jax import lax
from jax.experimental import pallas as pl
from jax.experimental.pallas import tpu as pltpu
from jax.experimental.pallas import tpu_sc as plsc

VOCAB = 100000
EMBED = 128
WINDOW = 2048
BATCH = 4
SEQ = 2048

SCALE = float(np.sqrt(float(EMBED)))

NUM_CORES = 2
NUM_SUBCORES = 16
NW = NUM_CORES * NUM_SUBCORES
PPW = SEQ // NW
LANES = 16
NVEC = EMBED // LANES
HALF = EMBED // 2
NEG_LN = -math.log(10000.0) / HALF
WID_BITS = 5

_MESH = plsc.VectorSubcoreMesh(
    core_axis_name="c", subcore_axis_name="s",
    num_cores=NUM_CORES, num_subcores=NUM_SUBCORES,
)


@functools.partial(
    pl.kernel,
    out_type=jax.ShapeDtypeStruct((BATCH, SEQ, EMBED), jnp.float32),
    mesh=_MESH,
    scratch_types=[
        pltpu.VMEM((BATCH, PPW), jnp.int32),
        pltpu.VMEM((BATCH, PPW, EMBED), jnp.float32),
        pltpu.VMEM((PPW, EMBED), jnp.float32),
        pltpu.VMEM((4, EMBED), jnp.float32),
        pltpu.SemaphoreType.DMA,
        pltpu.SemaphoreType.DMA,
    ],
)
def _sc_embed(x_hbm, table_hbm, out_hbm, idx_v, rows_v, pe_v, bas_v, sem, osem):
    wid = lax.axis_index("s") * NUM_CORES + lax.axis_index("c")
    p0 = wid * PPW
    cb = lax.div(wid, 2)
    coff = lax.rem(wid, 2) * PPW

    idx_cps = [
        pltpu.async_copy(x_hbm.at[cb, b, pl.ds(coff, PPW)], idx_v.at[b], osem)
        for b in range(BATCH)
    ]
    for b in range(BATCH):
        idx_cps[b].wait()
        pltpu.async_copy(table_hbm.at[idx_v.at[b]], rows_v.at[b], sem)

    iota = lax.convert_element_type(lax.iota(jnp.int32, LANES), jnp.float32)
    bitfs = [
        lax.convert_element_type(
            lax.bitwise_and(lax.shift_right_logical(wid, k), 1), jnp.float32
        )
        for k in range(WID_BITS)
    ]

    def bas_body(k, carry):
        jv = iota + lax.convert_element_type(
            lax.rem(k, NVEC // 2) * LANES, jnp.float32
        )
        r = jnp.exp(jv * NEG_LN)
        u = r * r
        s1 = r * (1.0 + u * (-1.0 / 6 + u * (1.0 / 120
             + u * (-1.0 / 5040 + u * (1.0 / 362880)))))
        c1 = 1.0 + u * (-0.5 + u * (1.0 / 24 + u * (-1.0 / 720
             + u * (1.0 / 40320 + u * (-1.0 / 3628800)))))
        sl = pl.ds(k * LANES, LANES)
        bas_v[0, sl] = s1
        bas_v[1, sl] = c1
        s, c = s1, c1
        for _ in range(6):
            s, c = 2.0 * s * c, c * c - s * s
        S = iota * 0.0
        C = S + 1.0
        for kb in range(WID_BITS):
            sKb = bitfs[kb] * s
            cKb = 1.0 + bitfs[kb] * (c - 1.0)
            S, C = S * cKb + C * sKb, C * cKb - S * sKb
            if kb + 1 < WID_BITS:
                s, c = 2.0 * s * c, c * c - s * s
        bas_v[2, sl] = S
        bas_v[3, sl] = C
        return carry

    lax.fori_loop(0, NVEC, bas_body, 0)

    sr = [bas_v[0, pl.ds(j * LANES, LANES)] for j in range(NVEC)]
    cr = [bas_v[1, pl.ds(j * LANES, LANES)] for j in range(NVEC)]
    svec = [bas_v[2, pl.ds(j * LANES, LANES)] for j in range(NVEC)]
    cvec = [bas_v[3, pl.ds(j * LANES, LANES)] for j in range(NVEC)]

    def pe_body(q, carry):
        s = carry[:NVEC]
        c = carry[NVEC:]
        for j in range(NVEC):
            sl = pl.ds(j * LANES, LANES)
            pe_v[q, sl] = s[j] if j < NVEC // 2 else c[j]
        s_n = [s[j] * cr[j] + c[j] * sr[j] for j in range(NVEC)]
        c_n = [c[j] * cr[j] - s[j] * sr[j] for j in range(NVEC)]
        return tuple(s_n) + tuple(c_n)

    lax.fori_loop(0, PPW, pe_body, tuple(svec) + tuple(cvec))

    def b_body(b, carry):
        pltpu.make_async_copy(
            table_hbm.at[idx_v.at[0]], rows_v.at[0], sem
        ).wait()

        def body(q, c2):
            for j in range(NVEC):
                sl = pl.ds(j * LANES, LANES)
                rows_v[b, q, sl] = rows_v[b, q, sl] * SCALE + pe_v[q, sl]
            return c2

        lax.fori_loop(0, PPW, body, 0)
        pltpu.async_copy(rows_v.at[b], out_hbm.at[b].at[pl.ds(p0, PPW)], osem)
        return carry

    lax.fori_loop(0, BATCH, b_body, 0)
    for b in range(BATCH):
        pltpu.make_async_copy(
            rows_v.at[0], out_hbm.at[0].at[pl.ds(p0, PPW)], osem
        ).wait()


def kernel(x, table):
    x3 = jnp.transpose(
        x.astype(jnp.int32).reshape(BATCH, SEQ // 128, 128), (1, 0, 2)
    )
    return _sc_embed(x3, table)


# speedup vs baseline: 1.4543x; 1.4543x over previous
import functools

import numpy as np
import jax
import jax.numpy as jnp
from jax import lax
from jax.experimental import pallas as pl
from jax.experimental.pallas import tpu as pltpu
from jax.experimental.pallas import tpu_sc as plsc

VOCAB = 100000
EMBED = 128
WINDOW = 2048
BATCH = 4
SEQ = 2048

SCALE = float(np.sqrt(float(EMBED)))

NUM_CORES = 2
NUM_SUBCORES = 16
NW = NUM_CORES * NUM_SUBCORES
PPW = SEQ // NW
LANES = 16
NVEC = EMBED // LANES
HALF = EMBED // 2


def _pe_tables() -> np.ndarray:
    rates = 1.0 / 10000 ** (np.arange(HALF) / HALF)
    r = np.concatenate([rates, rates])
    p0 = (np.arange(NW) * PPW)[:, np.newaxis]
    tab = np.stack(
        [
            np.sin(p0 * r),
            np.cos(p0 * r),
            np.broadcast_to(np.cos(r), (NW, EMBED)),
            np.broadcast_to(np.sin(r), (NW, EMBED)),
        ],
        axis=1,
    )
    return np.ascontiguousarray(tab).astype(np.float32)


_TAB_NP = _pe_tables()

_MESH = plsc.VectorSubcoreMesh(
    core_axis_name="c", subcore_axis_name="s",
    num_cores=NUM_CORES, num_subcores=NUM_SUBCORES,
)


@functools.partial(
    pl.kernel,
    out_type=jax.ShapeDtypeStruct((BATCH, SEQ, EMBED), jnp.float32),
    mesh=_MESH,
    scratch_types=[
        pltpu.VMEM((BATCH, PPW), jnp.int32),
        pltpu.VMEM((BATCH, PPW, EMBED), jnp.float32),
        pltpu.VMEM((PPW, EMBED), jnp.float32),
        pltpu.VMEM((4, EMBED), jnp.float32),
        pltpu.SemaphoreType.DMA,
        pltpu.SemaphoreType.DMA,
    ],
)
def _sc_embed(x_hbm, table_hbm, tab_hbm, out_hbm, idx_v, rows_v, pe_v, ab_v,
              sem, osem):
    wid = lax.axis_index("s") * NUM_CORES + lax.axis_index("c")
    p0 = wid * PPW
    cb = lax.div(wid, 2)
    coff = lax.rem(wid, 2) * PPW

    tab_cp = pltpu.async_copy(tab_hbm.at[wid], ab_v, osem)
    idx_cps = [
        pltpu.async_copy(x_hbm.at[cb, b, pl.ds(coff, PPW)], idx_v.at[b], sem)
        for b in range(BATCH)
    ]
    gathers = []
    for b in range(BATCH):
        idx_cps[b].wait()
        gathers.append(
            pltpu.async_copy(table_hbm.at[idx_v.at[b]], rows_v.at[b], sem)
        )
    tab_cp.wait()

    svec = [ab_v[0, pl.ds(j * LANES, LANES)] for j in range(NVEC)]
    cvec = [ab_v[1, pl.ds(j * LANES, LANES)] for j in range(NVEC)]
    cr = [ab_v[2, pl.ds(j * LANES, LANES)] for j in range(NVEC)]
    sr = [ab_v[3, pl.ds(j * LANES, LANES)] for j in range(NVEC)]

    def pe_body(q, carry):
        s = carry[:NVEC]
        c = carry[NVEC:]
        for j in range(NVEC):
            sl = pl.ds(j * LANES, LANES)
            pe_v[q, sl] = s[j] if j < NVEC // 2 else c[j]
        s_n = [s[j] * cr[j] + c[j] * sr[j] for j in range(NVEC)]
        c_n = [c[j] * cr[j] - s[j] * sr[j] for j in range(NVEC)]
        return tuple(s_n) + tuple(c_n)

    lax.fori_loop(0, PPW, pe_body, tuple(svec) + tuple(cvec))

    outs = []
    for b in range(BATCH):
        gathers[b].wait()

        def body(q, carry, b=b):
            for j in range(NVEC):
                sl = pl.ds(j * LANES, LANES)
                rows_v[b, q, sl] = rows_v[b, q, sl] * SCALE + pe_v[q, sl]
            return carry

        lax.fori_loop(0, PPW, body, 0)
        outs.append(
            pltpu.async_copy(rows_v.at[b], out_hbm.at[b].at[pl.ds(p0, PPW)], osem)
        )
    for o in outs:
        o.wait()


def kernel(x, table):
    x3 = jnp.transpose(
        x.astype(jnp.int32).reshape(BATCH, SEQ // 128, 128), (1, 0, 2)
    )
    return _sc_embed(x3, table, jnp.asarray(_TAB_NP))
